# Initial kernel scaffold; baseline (speedup 1.0000x reference)
import jax
import jax.numpy as jnp
from jax.experimental import pallas as pl


def kernel(fea, ring_n):
    raise NotImplementedError("write your pallas kernel here")


# speedup vs baseline: 19.5655x; 19.5655x over previous
import functools

import jax
import jax.numpy as jnp
from jax import lax
from jax.experimental import pallas as pl
from jax.experimental.pallas import tpu as pltpu
from jax.experimental.pallas import tpu_sc as plsc

M, C, F = 4, 128, 50000
K = 3
NC, NS, L = 2, 16, 16
NW = NC * NS
ROWS_PER_W = (M * C) // NW
W_PER_MESH = C // ROWS_PER_W
FC = 10000
NCHUNK = F // FC
NVEC = FC // L


def _sc_body(fea_hbm, ring_hbm, out_hbm, fea_buf, idx0_buf, idx1_buf,
             idx2_buf, out_buf):
    cid = lax.axis_index("c")
    sid = lax.axis_index("s")
    wid = cid * NS + sid
    m = wid // W_PER_MESH
    c0 = (wid % W_PER_MESH) * ROWS_PER_W

    def row_body(r, carry):
        row = m * C + c0 + r
        pltpu.sync_copy(fea_hbm.at[pl.ds(row * F, F)], fea_buf)

        def chunk_body(fc, carry):
            f0 = fc * FC
            for k, buf in enumerate((idx0_buf, idx1_buf, idx2_buf)):
                pltpu.sync_copy(
                    ring_hbm.at[pl.ds((m * K + k) * F + f0, FC)], buf)

            def vec_body(j, carry):
                v = fea_buf[pl.ds(f0 + j * L, L)]
                i0 = idx0_buf[pl.ds(j * L, L)]
                i1 = idx1_buf[pl.ds(j * L, L)]
                i2 = idx2_buf[pl.ds(j * L, L)]
                g0 = plsc.load_gather(fea_buf, [i0])
                g1 = plsc.load_gather(fea_buf, [i1])
                g2 = plsc.load_gather(fea_buf, [i2])
                out_buf[pl.ds(j * L, L)] = jnp.maximum(
                    jnp.maximum(v, g0), jnp.maximum(g1, g2))
                return carry

            lax.fori_loop(0, NVEC, vec_body, 0)
            pltpu.sync_copy(out_buf, out_hbm.at[pl.ds(row * F + f0, FC)])
            return carry

        lax.fori_loop(0, NCHUNK, chunk_body, 0)
        return carry

    lax.fori_loop(0, ROWS_PER_W, row_body, 0)


_sc_pool = functools.partial(
    pl.kernel,
    mesh=plsc.VectorSubcoreMesh(core_axis_name="c", subcore_axis_name="s"),
    compiler_params=pltpu.CompilerParams(needs_layout_passes=False),
    out_type=jax.ShapeDtypeStruct((M * C * F,), jnp.float32),
    scratch_types=[
        pltpu.VMEM((F,), jnp.float32),
        pltpu.VMEM((FC,), jnp.int32),
        pltpu.VMEM((FC,), jnp.int32),
        pltpu.VMEM((FC,), jnp.int32),
        pltpu.VMEM((FC,), jnp.float32),
    ],
)(_sc_body)


def kernel(fea, ring_n):
    ring_t = jnp.transpose(ring_n, (0, 2, 1)).reshape(-1)
    return _sc_pool(fea.reshape(-1), ring_t).reshape(M, C, F)


# speedup vs baseline: 23.4747x; 1.1998x over previous
import functools

import jax
import jax.numpy as jnp
from jax import lax
from jax.experimental import pallas as pl
from jax.experimental.pallas import tpu as pltpu
from jax.experimental.pallas import tpu_sc as plsc

M, C, F = 4, 128, 50000
K = 3
NC, NS, L = 2, 16, 16
NW = NC * NS
ROWS_PER_W = (M * C) // NW
W_PER_MESH = C // ROWS_PER_W

FP = 50048
NG = FP // (2 * L)
NCHUNK = 4
GC = NG // NCHUNK
FC = GC * 2 * L
IDXC = K * GC * L
F_LAST = F - (NCHUNK - 1) * FC


def _sc_body(fea_hbm, ring_hbm, out_hbm, fea_buf, idx_a, idx_b, out_a,
             out_b, sem_ia, sem_ib, sem_oa, sem_ob):
    cid = lax.axis_index("c")
    sid = lax.axis_index("s")
    wid = cid * NS + sid
    m = wid // W_PER_MESH
    c0 = (wid % W_PER_MESH) * ROWS_PER_W
    ring_base = m * NCHUNK * IDXC

    bufs = [(idx_a, out_a, sem_ia, sem_oa), (idx_b, out_b, sem_ib, sem_ob)]

    def row_body(r, carry):
        row = m * C + c0 + r
        pltpu.sync_copy(fea_hbm.at[pl.ds(row * F, F)], fea_buf.at[pl.ds(0, F)])

        h_idx = [None] * NCHUNK
        h_out = [None] * NCHUNK
        h_idx[0] = pltpu.async_copy(
            ring_hbm.at[pl.ds(ring_base, IDXC)], idx_a, sem_ia)

        for fc in range(NCHUNK):
            ib, ob, _, s_o = bufs[fc % 2]
            if fc + 1 < NCHUNK:
                nib, _, s_ni, _ = bufs[(fc + 1) % 2]
                h_idx[fc + 1] = pltpu.async_copy(
                    ring_hbm.at[pl.ds(ring_base + (fc + 1) * IDXC, IDXC)],
                    nib, s_ni)
            h_idx[fc].wait()
            if fc >= 2:
                h_out[fc - 2].wait()
            cb = fc * FC

            def group_body(g, carry, ib=ib, ob=ob, cb=cb):
                x0 = ib[pl.ds(g * L, L)]
                x1 = ib[pl.ds(GC * L + g * L, L)]
                x2 = ib[pl.ds(2 * GC * L + g * L, L)]
                lo0 = x0 & 0xFFFF
                lo1 = x1 & 0xFFFF
                lo2 = x2 & 0xFFFF
                hi0 = lax.shift_right_logical(x0, 16)
                hi1 = lax.shift_right_logical(x1, 16)
                hi2 = lax.shift_right_logical(x2, 16)
                v_lo = fea_buf[pl.ds(cb + g * 2 * L, L)]
                v_hi = fea_buf[pl.ds(cb + g * 2 * L + L, L)]
                g0 = plsc.load_gather(fea_buf, [lo0])
                g1 = plsc.load_gather(fea_buf, [lo1])
                g2 = plsc.load_gather(fea_buf, [lo2])
                g3 = plsc.load_gather(fea_buf, [hi0])
                g4 = plsc.load_gather(fea_buf, [hi1])
                g5 = plsc.load_gather(fea_buf, [hi2])
                ob[pl.ds(g * 2 * L, L)] = jnp.maximum(
                    jnp.maximum(v_lo, g0), jnp.maximum(g1, g2))
                ob[pl.ds(g * 2 * L + L, L)] = jnp.maximum(
                    jnp.maximum(v_hi, g3), jnp.maximum(g4, g5))
                return carry

            lax.fori_loop(0, GC, group_body, 0)
            n_out = FC if fc + 1 < NCHUNK else F_LAST
            h_out[fc] = pltpu.async_copy(
                ob.at[pl.ds(0, n_out)],
                out_hbm.at[pl.ds(row * F + cb, n_out)], s_o)

        h_out[NCHUNK - 2].wait()
        h_out[NCHUNK - 1].wait()
        return carry

    lax.fori_loop(0, ROWS_PER_W, row_body, 0)


_sc_pool = functools.partial(
    pl.kernel,
    mesh=plsc.VectorSubcoreMesh(core_axis_name="c", subcore_axis_name="s"),
    compiler_params=pltpu.CompilerParams(needs_layout_passes=False),
    out_type=jax.ShapeDtypeStruct((M * C * F,), jnp.float32),
    scratch_types=[
        pltpu.VMEM((FP,), jnp.float32),
        pltpu.VMEM((IDXC,), jnp.int32),
        pltpu.VMEM((IDXC,), jnp.int32),
        pltpu.VMEM((FC,), jnp.float32),
        pltpu.VMEM((FC,), jnp.float32),
        pltpu.SemaphoreType.DMA,
        pltpu.SemaphoreType.DMA,
        pltpu.SemaphoreType.DMA,
        pltpu.SemaphoreType.DMA,
    ],
)(_sc_body)


def kernel(fea, ring_n):
    ring_t = jnp.transpose(ring_n, (0, 2, 1))
    ring_p = jnp.pad(ring_t, ((0, 0), (0, 0), (0, FP - F)))
    pairs = ring_p.reshape(M, K, NG, 2, L)
    packed = pairs[:, :, :, 0, :] | (pairs[:, :, :, 1, :] << 16)
    packed = packed.reshape(M, K, NCHUNK, GC, L).transpose(0, 2, 1, 3, 4)
    return _sc_pool(fea.reshape(-1), packed.reshape(-1)).reshape(M, C, F)


# speedup vs baseline: 28.7405x; 1.2243x over previous
import functools

import jax
import jax.numpy as jnp
from jax import lax
from jax.experimental import pallas as pl
from jax.experimental.pallas import tpu as pltpu
from jax.experimental.pallas import tpu_sc as plsc

M, C, F = 4, 128, 50000
K = 3
NC, NS, L = 2, 16, 16
NW = NC * NS
ROWS_PER_W = (M * C) // NW
W_PER_MESH = C // ROWS_PER_W

FP = 50048
NG = FP // (2 * L)
NCHUNK = 4
GC = NG // NCHUNK
FC = GC * 2 * L
IDXC = K * GC * L
F_LAST = F - (NCHUNK - 1) * FC


def _sc_body(fea_hbm, ring_hbm, out_hbm, fea_buf, idx_a, idx_b, out_a,
             out_b, sem_ia, sem_ib, sem_oa, sem_ob):
    cid = lax.axis_index("c")
    sid = lax.axis_index("s")
    wid = cid * NS + sid
    m = wid // W_PER_MESH
    c0 = (wid % W_PER_MESH) * ROWS_PER_W
    ring_base = m * NCHUNK * IDXC

    bufs = [(idx_a, out_a, sem_ia, sem_oa), (idx_b, out_b, sem_ib, sem_ob)]

    def row_body(r, carry):
        row = m * C + c0 + r
        pltpu.sync_copy(fea_hbm.at[pl.ds(row * F, F)], fea_buf.at[pl.ds(0, F)])

        h_idx = [None] * NCHUNK
        h_out = [None] * NCHUNK
        h_idx[0] = pltpu.async_copy(
            ring_hbm.at[pl.ds(ring_base, IDXC)], idx_a, sem_ia)

        for fc in range(NCHUNK):
            ib, ob, _, s_o = bufs[fc % 2]
            if fc + 1 < NCHUNK:
                nib, _, s_ni, _ = bufs[(fc + 1) % 2]
                h_idx[fc + 1] = pltpu.async_copy(
                    ring_hbm.at[pl.ds(ring_base + (fc + 1) * IDXC, IDXC)],
                    nib, s_ni)
            h_idx[fc].wait()
            if fc >= 2:
                h_out[fc - 2].wait()
            cb = fc * FC

            @plsc.parallel_loop(0, GC, 1, unroll=2)
            def group_body(g, ib=ib, ob=ob, cb=cb):
                x0 = ib[pl.ds(g * L, L)]
                x1 = ib[pl.ds(GC * L + g * L, L)]
                x2 = ib[pl.ds(2 * GC * L + g * L, L)]
                lo0 = x0 & 0xFFFF
                lo1 = x1 & 0xFFFF
                lo2 = x2 & 0xFFFF
                hi0 = lax.shift_right_logical(x0, 16)
                hi1 = lax.shift_right_logical(x1, 16)
                hi2 = lax.shift_right_logical(x2, 16)
                v_lo = fea_buf[pl.ds(cb + g * 2 * L, L)]
                v_hi = fea_buf[pl.ds(cb + g * 2 * L + L, L)]
                g0 = plsc.load_gather(fea_buf, [lo0])
                g1 = plsc.load_gather(fea_buf, [lo1])
                g2 = plsc.load_gather(fea_buf, [lo2])
                g3 = plsc.load_gather(fea_buf, [hi0])
                g4 = plsc.load_gather(fea_buf, [hi1])
                g5 = plsc.load_gather(fea_buf, [hi2])
                ob[pl.ds(g * 2 * L, L)] = jnp.maximum(
                    jnp.maximum(v_lo, g0), jnp.maximum(g1, g2))
                ob[pl.ds(g * 2 * L + L, L)] = jnp.maximum(
                    jnp.maximum(v_hi, g3), jnp.maximum(g4, g5))

            n_out = FC if fc + 1 < NCHUNK else F_LAST
            h_out[fc] = pltpu.async_copy(
                ob.at[pl.ds(0, n_out)],
                out_hbm.at[pl.ds(row * F + cb, n_out)], s_o)

        h_out[NCHUNK - 2].wait()
        h_out[NCHUNK - 1].wait()
        return carry

    lax.fori_loop(0, ROWS_PER_W, row_body, 0)


_sc_pool = functools.partial(
    pl.kernel,
    mesh=plsc.VectorSubcoreMesh(core_axis_name="c", subcore_axis_name="s"),
    compiler_params=pltpu.CompilerParams(needs_layout_passes=False),
    out_type=jax.ShapeDtypeStruct((M * C * F,), jnp.float32),
    scratch_types=[
        pltpu.VMEM((FP,), jnp.float32),
        pltpu.VMEM((IDXC,), jnp.int32),
        pltpu.VMEM((IDXC,), jnp.int32),
        pltpu.VMEM((FC,), jnp.float32),
        pltpu.VMEM((FC,), jnp.float32),
        pltpu.SemaphoreType.DMA,
        pltpu.SemaphoreType.DMA,
        pltpu.SemaphoreType.DMA,
        pltpu.SemaphoreType.DMA,
    ],
)(_sc_body)


def kernel(fea, ring_n):
    ring_t = jnp.transpose(ring_n, (0, 2, 1))
    ring_p = jnp.pad(ring_t, ((0, 0), (0, 0), (0, FP - F)))
    pairs = ring_p.reshape(M, K, NG, 2, L)
    packed = pairs[:, :, :, 0, :] | (pairs[:, :, :, 1, :] << 16)
    packed = packed.reshape(M, K, NCHUNK, GC, L).transpose(0, 2, 1, 3, 4)
    return _sc_pool(fea.reshape(-1), packed.reshape(-1)).reshape(M, C, F)


# speedup vs baseline: 30.1502x; 1.0490x over previous
import functools

import jax
import jax.numpy as jnp
from jax import lax
from jax.experimental import pallas as pl
from jax.experimental.pallas import tpu as pltpu
from jax.experimental.pallas import tpu_sc as plsc

M, C, F = 4, 128, 50000
K = 3
NC, NS, L = 2, 16, 16
NW = NC * NS
ROWS_PER_W = (M * C) // NW
W_PER_MESH = C // ROWS_PER_W

FP = 50048
NG = FP // (2 * L)
NCHUNK = 4
GC = NG // NCHUNK
FC = GC * 2 * L
F_LAST = F - (NCHUNK - 1) * FC


def _sc_body(fea_hbm, ring_hbm, out_hbm, fea_buf, ia0, ia1, ia2, ib0, ib1,
             ib2, out_a, out_b, sem_ia, sem_ib, sem_oa, sem_ob):
    cid = lax.axis_index("c")
    sid = lax.axis_index("s")
    wid = cid * NS + sid
    m = wid // W_PER_MESH
    c0 = (wid % W_PER_MESH) * ROWS_PER_W

    ibufs = [((ia0, ia1, ia2), sem_ia), ((ib0, ib1, ib2), sem_ib)]
    obufs = [(out_a, sem_oa), (out_b, sem_ob)]

    def idx_dma(fc):
        ibs, s_i = ibufs[fc % 2]
        return [
            pltpu.async_copy(
                ring_hbm.at[m, k, pl.ds(fc * GC, GC), :], ibs[k], s_i)
            for k in range(K)
        ]

    def row_body(r, carry):
        row = m * C + c0 + r
        pltpu.sync_copy(fea_hbm.at[pl.ds(row * F, F)], fea_buf.at[pl.ds(0, F)])

        h_idx = [None] * NCHUNK
        h_out = [None] * NCHUNK
        h_idx[0] = idx_dma(0)

        for fc in range(NCHUNK):
            ibs = ibufs[fc % 2][0]
            ob, s_o = obufs[fc % 2]
            if fc + 1 < NCHUNK:
                h_idx[fc + 1] = idx_dma(fc + 1)
            for h in h_idx[fc]:
                h.wait()
            if fc >= 2:
                h_out[fc - 2].wait()
            cb = fc * FC

            @plsc.parallel_loop(0, GC, 1, unroll=2)
            def group_body(g, ibs=ibs, ob=ob, cb=cb):
                x0 = ibs[0][g, :]
                x1 = ibs[1][g, :]
                x2 = ibs[2][g, :]
                lo0 = x0 & 0xFFFF
                lo1 = x1 & 0xFFFF
                lo2 = x2 & 0xFFFF
                hi0 = lax.shift_right_logical(x0, 16)
                hi1 = lax.shift_right_logical(x1, 16)
                hi2 = lax.shift_right_logical(x2, 16)
                v_lo = fea_buf[pl.ds(cb + g * 2 * L, L)]
                v_hi = fea_buf[pl.ds(cb + g * 2 * L + L, L)]
                g0 = plsc.load_gather(fea_buf, [lo0])
                g1 = plsc.load_gather(fea_buf, [lo1])
                g2 = plsc.load_gather(fea_buf, [lo2])
                g3 = plsc.load_gather(fea_buf, [hi0])
                g4 = plsc.load_gather(fea_buf, [hi1])
                g5 = plsc.load_gather(fea_buf, [hi2])
                ob[pl.ds(g * 2 * L, L)] = jnp.maximum(
                    jnp.maximum(v_lo, g0), jnp.maximum(g1, g2))
                ob[pl.ds(g * 2 * L + L, L)] = jnp.maximum(
                    jnp.maximum(v_hi, g3), jnp.maximum(g4, g5))

            n_out = FC if fc + 1 < NCHUNK else F_LAST
            h_out[fc] = pltpu.async_copy(
                ob.at[pl.ds(0, n_out)],
                out_hbm.at[pl.ds(row * F + cb, n_out)], s_o)

        h_out[NCHUNK - 2].wait()
        h_out[NCHUNK - 1].wait()
        return carry

    lax.fori_loop(0, ROWS_PER_W, row_body, 0)


_sc_pool = functools.partial(
    pl.kernel,
    mesh=plsc.VectorSubcoreMesh(core_axis_name="c", subcore_axis_name="s"),
    compiler_params=pltpu.CompilerParams(
        needs_layout_passes=False, use_tc_tiling_on_sc=False),
    out_type=jax.ShapeDtypeStruct((M * C * F,), jnp.float32),
    scratch_types=[
        pltpu.VMEM((FP,), jnp.float32),
        pltpu.VMEM((GC, L), jnp.int32),
        pltpu.VMEM((GC, L), jnp.int32),
        pltpu.VMEM((GC, L), jnp.int32),
        pltpu.VMEM((GC, L), jnp.int32),
        pltpu.VMEM((GC, L), jnp.int32),
        pltpu.VMEM((GC, L), jnp.int32),
        pltpu.VMEM((FC,), jnp.float32),
        pltpu.VMEM((FC,), jnp.float32),
        pltpu.SemaphoreType.DMA,
        pltpu.SemaphoreType.DMA,
        pltpu.SemaphoreType.DMA,
        pltpu.SemaphoreType.DMA,
    ],
)(_sc_body)


def kernel(fea, ring_n):
    ring_t = jnp.transpose(ring_n, (0, 2, 1))
    ring_p = jnp.pad(ring_t, ((0, 0), (0, 0), (0, FP - F)))
    pairs = ring_p.reshape(M, K, NG, 2, L)
    packed = pairs[:, :, :, 0, :] | (pairs[:, :, :, 1, :] << 16)
    return _sc_pool(fea.reshape(-1), packed).reshape(M, C, F)


# speedup vs baseline: 30.7298x; 1.0192x over previous
import functools

import jax
import jax.numpy as jnp
from jax import lax
from jax.experimental import pallas as pl
from jax.experimental.pallas import tpu as pltpu
from jax.experimental.pallas import tpu_sc as plsc

M, C, F = 4, 128, 50000
K = 3
NC, NS, L = 2, 16, 16
NW = NC * NS
ROWS_PER_W = (M * C) // NW
W_PER_MESH = C // ROWS_PER_W

FP = 50048
NG = FP // (2 * L)
NCHUNK = 4
GC = NG // NCHUNK
FC = GC * 2 * L
F_LAST = F - (NCHUNK - 1) * FC


def _sc_body(fea_hbm, ring_hbm, out_hbm, fea_buf, ia0, ia1, ia2, ib0, ib1,
             ib2, out_a, out_b, sem_ia, sem_ib, sem_oa, sem_ob):
    cid = lax.axis_index("c")
    sid = lax.axis_index("s")
    wid = cid * NS + sid
    m = wid // W_PER_MESH
    c0 = (wid % W_PER_MESH) * ROWS_PER_W

    ibufs = [((ia0, ia1, ia2), sem_ia), ((ib0, ib1, ib2), sem_ib)]
    obufs = [(out_a, sem_oa), (out_b, sem_ob)]

    iota2 = lax.iota(jnp.int32, L) * 2

    def idx_dma(fc):
        ibs, s_i = ibufs[fc % 2]
        return [
            pltpu.async_copy(
                ring_hbm.at[m, k, pl.ds(fc * GC * L, GC * L)], ibs[k], s_i)
            for k in range(K)
        ]

    def row_body(r, carry):
        row = m * C + c0 + r
        pltpu.sync_copy(fea_hbm.at[pl.ds(row * F, F)], fea_buf.at[pl.ds(0, F)])

        h_idx = [None] * NCHUNK
        h_out = [None] * NCHUNK
        h_idx[0] = idx_dma(0)

        for fc in range(NCHUNK):
            ibs = ibufs[fc % 2][0]
            ob, s_o = obufs[fc % 2]
            if fc + 1 < NCHUNK:
                h_idx[fc + 1] = idx_dma(fc + 1)
            for h in h_idx[fc]:
                h.wait()
            if fc >= 2:
                h_out[fc - 2].wait()
            cb = fc * FC

            @plsc.parallel_loop(0, GC, 1, unroll=2)
            def group_body(g, ibs=ibs, ob=ob, cb=cb):
                x0 = ibs[0][pl.ds(g * L, L)]
                x1 = ibs[1][pl.ds(g * L, L)]
                x2 = ibs[2][pl.ds(g * L, L)]
                lo0 = x0 & 0xFFFF
                lo1 = x1 & 0xFFFF
                lo2 = x2 & 0xFFFF
                hi0 = lax.shift_right_logical(x0, 16)
                hi1 = lax.shift_right_logical(x1, 16)
                hi2 = lax.shift_right_logical(x2, 16)
                ev = iota2 + (cb + g * 2 * L)
                od = ev + 1
                v_ev = plsc.load_gather(fea_buf, [ev])
                v_od = plsc.load_gather(fea_buf, [od])
                g0 = plsc.load_gather(fea_buf, [lo0])
                g1 = plsc.load_gather(fea_buf, [lo1])
                g2 = plsc.load_gather(fea_buf, [lo2])
                g3 = plsc.load_gather(fea_buf, [hi0])
                g4 = plsc.load_gather(fea_buf, [hi1])
                g5 = plsc.load_gather(fea_buf, [hi2])
                oev = iota2 + g * 2 * L
                plsc.store_scatter(ob, [oev], jnp.maximum(
                    jnp.maximum(v_ev, g0), jnp.maximum(g1, g2)))
                plsc.store_scatter(ob, [oev + 1], jnp.maximum(
                    jnp.maximum(v_od, g3), jnp.maximum(g4, g5)))

            n_out = FC if fc + 1 < NCHUNK else F_LAST
            h_out[fc] = pltpu.async_copy(
                ob.at[pl.ds(0, n_out)],
                out_hbm.at[pl.ds(row * F + cb, n_out)], s_o)

        h_out[NCHUNK - 2].wait()
        h_out[NCHUNK - 1].wait()
        return carry

    lax.fori_loop(0, ROWS_PER_W, row_body, 0)


_sc_pool = functools.partial(
    pl.kernel,
    mesh=plsc.VectorSubcoreMesh(core_axis_name="c", subcore_axis_name="s"),
    compiler_params=pltpu.CompilerParams(
        needs_layout_passes=False, use_tc_tiling_on_sc=False),
    out_type=jax.ShapeDtypeStruct((M * C * F,), jnp.float32),
    scratch_types=[
        pltpu.VMEM((FP,), jnp.float32),
        pltpu.VMEM((GC * L,), jnp.int32),
        pltpu.VMEM((GC * L,), jnp.int32),
        pltpu.VMEM((GC * L,), jnp.int32),
        pltpu.VMEM((GC * L,), jnp.int32),
        pltpu.VMEM((GC * L,), jnp.int32),
        pltpu.VMEM((GC * L,), jnp.int32),
        pltpu.VMEM((FC,), jnp.float32),
        pltpu.VMEM((FC,), jnp.float32),
        pltpu.SemaphoreType.DMA,
        pltpu.SemaphoreType.DMA,
        pltpu.SemaphoreType.DMA,
        pltpu.SemaphoreType.DMA,
    ],
)(_sc_body)


def kernel(fea, ring_n):
    ring_p = jnp.pad(ring_n, ((0, 0), (0, FP - F), (0, 0)))
    packed = ring_p[:, 0::2, :] | (ring_p[:, 1::2, :] << 16)
    packed = jnp.transpose(packed, (0, 2, 1))
    return _sc_pool(fea.reshape(-1), packed).reshape(M, C, F)


# speedup vs baseline: 30.7946x; 1.0021x over previous
import functools

import jax
import jax.numpy as jnp
from jax import lax
from jax.experimental import pallas as pl
from jax.experimental.pallas import tpu as pltpu
from jax.experimental.pallas import tpu_sc as plsc

M, C, F = 4, 128, 50000
K = 3
NC, NS, L = 2, 16, 16
NW = NC * NS
ROWS_PER_W = (M * C) // NW
W_PER_MESH = C // ROWS_PER_W

FP = 50048
NG = FP // (2 * L)
NCHUNK = 4
GC = NG // NCHUNK
FC = GC * 2 * L
F_LAST = F - (NCHUNK - 1) * FC


def _sc_body(fea_hbm, ring_hbm, out_hbm, fea_buf, ia0, ia1, ia2, ib0, ib1,
             ib2, out_a, out_b, sem_ia, sem_ib, sem_oa, sem_ob):
    cid = lax.axis_index("c")
    sid = lax.axis_index("s")
    wid = cid * NS + sid
    m = wid // W_PER_MESH
    c0 = (wid % W_PER_MESH) * ROWS_PER_W

    ibufs = [((ia0, ia1, ia2), sem_ia), ((ib0, ib1, ib2), sem_ib)]
    obufs = [(out_a, sem_oa), (out_b, sem_ob)]

    iota2 = lax.iota(jnp.int32, L) * 2

    def idx_dma(fc):
        ibs, s_i = ibufs[fc % 2]
        return [
            pltpu.async_copy(
                ring_hbm.at[m, k, pl.ds(fc * GC * L, GC * L)], ibs[k], s_i)
            for k in range(K)
        ]

    def row_body(r, carry):
        row = m * C + c0 + r
        pltpu.sync_copy(fea_hbm.at[pl.ds(row * F, F)], fea_buf.at[pl.ds(0, F)])

        h_idx = [None] * NCHUNK
        h_out = [None] * NCHUNK
        h_idx[0] = idx_dma(0)

        for fc in range(NCHUNK):
            ibs = ibufs[fc % 2][0]
            ob, s_o = obufs[fc % 2]
            if fc + 1 < NCHUNK:
                h_idx[fc + 1] = idx_dma(fc + 1)
            for h in h_idx[fc]:
                h.wait()
            if fc >= 2:
                h_out[fc - 2].wait()
            cb = fc * FC

            @plsc.parallel_loop(0, GC, 1, unroll=4)
            def group_body(g, ibs=ibs, ob=ob, cb=cb):
                x0 = ibs[0][pl.ds(g * L, L)]
                x1 = ibs[1][pl.ds(g * L, L)]
                x2 = ibs[2][pl.ds(g * L, L)]
                lo0 = x0 & 0xFFFF
                lo1 = x1 & 0xFFFF
                lo2 = x2 & 0xFFFF
                hi0 = lax.shift_right_logical(x0, 16)
                hi1 = lax.shift_right_logical(x1, 16)
                hi2 = lax.shift_right_logical(x2, 16)
                ev = iota2 + (cb + g * 2 * L)
                od = ev + 1
                v_ev = plsc.load_gather(fea_buf, [ev])
                v_od = plsc.load_gather(fea_buf, [od])
                g0 = plsc.load_gather(fea_buf, [lo0])
                g1 = plsc.load_gather(fea_buf, [lo1])
                g2 = plsc.load_gather(fea_buf, [lo2])
                g3 = plsc.load_gather(fea_buf, [hi0])
                g4 = plsc.load_gather(fea_buf, [hi1])
                g5 = plsc.load_gather(fea_buf, [hi2])
                oev = iota2 + g * 2 * L
                plsc.store_scatter(ob, [oev], jnp.maximum(
                    jnp.maximum(v_ev, g0), jnp.maximum(g1, g2)))
                plsc.store_scatter(ob, [oev + 1], jnp.maximum(
                    jnp.maximum(v_od, g3), jnp.maximum(g4, g5)))

            n_out = FC if fc + 1 < NCHUNK else F_LAST
            h_out[fc] = pltpu.async_copy(
                ob.at[pl.ds(0, n_out)],
                out_hbm.at[pl.ds(row * F + cb, n_out)], s_o)

        h_out[NCHUNK - 2].wait()
        h_out[NCHUNK - 1].wait()
        return carry

    lax.fori_loop(0, ROWS_PER_W, row_body, 0)


_sc_pool = functools.partial(
    pl.kernel,
    mesh=plsc.VectorSubcoreMesh(core_axis_name="c", subcore_axis_name="s"),
    compiler_params=pltpu.CompilerParams(
        needs_layout_passes=False, use_tc_tiling_on_sc=False),
    out_type=jax.ShapeDtypeStruct((M * C * F,), jnp.float32),
    scratch_types=[
        pltpu.VMEM((FP,), jnp.float32),
        pltpu.VMEM((GC * L,), jnp.int32),
        pltpu.VMEM((GC * L,), jnp.int32),
        pltpu.VMEM((GC * L,), jnp.int32),
        pltpu.VMEM((GC * L,), jnp.int32),
        pltpu.VMEM((GC * L,), jnp.int32),
        pltpu.VMEM((GC * L,), jnp.int32),
        pltpu.VMEM((FC,), jnp.float32),
        pltpu.VMEM((FC,), jnp.float32),
        pltpu.SemaphoreType.DMA,
        pltpu.SemaphoreType.DMA,
        pltpu.SemaphoreType.DMA,
        pltpu.SemaphoreType.DMA,
    ],
)(_sc_body)


def kernel(fea, ring_n):
    ring_p = jnp.pad(ring_n, ((0, 0), (0, FP - F), (0, 0)))
    packed = ring_p[:, 0::2, :] | (ring_p[:, 1::2, :] << 16)
    packed = jnp.transpose(packed, (0, 2, 1))
    return _sc_pool(fea.reshape(-1), packed).reshape(M, C, F)
